# Initial kernel scaffold; baseline (speedup 1.0000x reference)
import jax
import jax.numpy as jnp
from jax.experimental import pallas as pl


def kernel(node_feat, score, batch, codebook, W, b):
    raise NotImplementedError("write your pallas kernel here")


# speedup vs baseline: 2.4689x; 2.4689x over previous
import functools

import jax
import jax.numpy as jnp
from jax import lax
from jax.experimental import pallas as pl
from jax.experimental.pallas import tpu as pltpu
from jax.experimental.pallas import tpu_sc as plsc

N = 10000
EMB = 256
K = 1024
G = 128
NC = 10
CW = 1.0

BN = 512
N1 = 10240
NBLK = N1 // BN

NTILES = 32
CHUNKS = 3
CB = 128
PER_TILE = CHUNKS * CB
N2 = NTILES * PER_TILE
GK = G * K
STRIPE = 2 * GK // 16


def _split3(v):
    h = v.astype(jnp.bfloat16)
    r = v - h.astype(jnp.float32)
    mid = r.astype(jnp.bfloat16)
    low = (r - mid.astype(jnp.float32)).astype(jnp.bfloat16)
    return h, mid, low


def _stage_a_body(x_ref, sv_ref, bt_ref, cb_ref,
                  idx_ref, p_ref, q_ref, cnt_ref, loss_ref):
    pid = pl.program_id(0)

    @pl.when(pid == 0)
    def _init():
        p_ref[...] = jnp.zeros_like(p_ref)
        q_ref[...] = jnp.zeros_like(q_ref)
        cnt_ref[...] = jnp.zeros_like(cnt_ref)
        loss_ref[...] = jnp.zeros_like(loss_ref)

    x = x_ref[...]
    cb = cb_ref[...]
    xc = lax.dot_general(x, cb, (((1,), (1,)), ((), ())),
                         preferred_element_type=jnp.float32)
    x2 = jnp.sum(x * x, axis=1, keepdims=True)
    csh, csm, csl = _split3(cb * cb)
    ones_row = jnp.ones((1, EMB), jnp.bfloat16)

    def odot(rhs):
        return lax.dot_general(ones_row, rhs, (((1,), (1,)), ((), ())),
                               preferred_element_type=jnp.float32)

    c2 = odot(csl) + odot(csm) + odot(csh)
    d = x2 - 2.0 * xc + c2
    m = jnp.min(d, axis=1, keepdims=True)
    kio = lax.broadcasted_iota(jnp.int32, (BN, K), 1)
    a = jnp.min(jnp.where(d == m, kio, K), axis=1, keepdims=True)
    idx_ref[...] = a

    score = sv_ref[:, 0:1]
    valid = sv_ref[:, 1:2]
    ohb = (lax.broadcasted_iota(jnp.int32, (BN, G), 1)
           == bt_ref[...]).astype(jnp.bfloat16)

    def tdot(lhs, rhs):
        return lax.dot_general(lhs, rhs, (((0,), (0,)), ((), ())),
                               preferred_element_type=jnp.float32)

    xh, xm, xl = _split3(x)
    sh, sm, sl = _split3(score)
    p_ref[...] += (tdot(ohb * sh, xl) + tdot(ohb * sm, xm)
                   + tdot(ohb * sl, xh) + tdot(ohb * sm, xh)
                   + tdot(ohb * sh, xm) + tdot(ohb * sh, xh))
    q_ref[...] += tdot(ohb, xl) + tdot(ohb, xm) + tdot(ohb, xh)
    cnt_ref[...] += tdot(ohb, valid.astype(jnp.bfloat16))
    loss_ref[...] += jnp.sum(m * valid).reshape(1, 1)


_stage_a = pl.pallas_call(
    _stage_a_body,
    grid=(NBLK,),
    in_specs=[
        pl.BlockSpec((BN, EMB), lambda i: (i, 0)),
        pl.BlockSpec((BN, 2), lambda i: (i, 0)),
        pl.BlockSpec((BN, 1), lambda i: (i, 0)),
        pl.BlockSpec((K, EMB), lambda i: (0, 0)),
    ],
    out_specs=[
        pl.BlockSpec((BN, 1), lambda i: (i, 0)),
        pl.BlockSpec((G, EMB), lambda i: (0, 0)),
        pl.BlockSpec((G, EMB), lambda i: (0, 0)),
        pl.BlockSpec((G, 1), lambda i: (0, 0)),
        pl.BlockSpec((1, 1), lambda i: (0, 0)),
    ],
    out_shape=[
        jax.ShapeDtypeStruct((N1, 1), jnp.int32),
        jax.ShapeDtypeStruct((G, EMB), jnp.float32),
        jax.ShapeDtypeStruct((G, EMB), jnp.float32),
        jax.ShapeDtypeStruct((G, 1), jnp.float32),
        jax.ShapeDtypeStruct((1, 1), jnp.float32),
    ],
    compiler_params=pltpu.CompilerParams(dimension_semantics=("arbitrary",)),
)


def _sc_stage_body(bt_hbm, ix_hbm, sw_hbm, vw_hbm, z_hbm, out_hbm,
                   bv, kv, fi, fi2, sw, vw, acc):
    cid = lax.axis_index("c")
    sid = lax.axis_index("s")
    tid = cid * 16 + sid
    pltpu.sync_copy(z_hbm, acc.at[pl.ds(sid * STRIPE, STRIPE)])
    pltpu.sync_copy(bt_hbm.at[tid], bv)
    pltpu.sync_copy(ix_hbm.at[tid], kv)
    pltpu.sync_copy(sw_hbm.at[tid], sw)
    pltpu.sync_copy(vw_hbm.at[tid], vw)
    for j in range(CHUNKS):
        for l in range(CB // 16):
            s_ = pl.ds(l * 16, 16)
            f = bv[j, s_] * K + kv[j, s_]
            fi[j, s_] = f
            fi2[j, s_] = f + GK
    plsc.subcore_barrier()
    for j in range(CHUNKS):
        pltpu.sync_copy(sw.at[j], acc.at[fi.at[j]], add=True)
        pltpu.sync_copy(vw.at[j], acc.at[fi2.at[j]], add=True)
    plsc.subcore_barrier()
    pltpu.sync_copy(acc.at[pl.ds(sid * STRIPE, STRIPE)],
                    out_hbm.at[cid, pl.ds(sid * STRIPE, STRIPE)])


@functools.cache
def _build_sc_stage():
    return functools.partial(
        pl.kernel,
        mesh=plsc.VectorSubcoreMesh(core_axis_name="c", subcore_axis_name="s"),
        out_type=jax.ShapeDtypeStruct((2, 2 * GK), jnp.float32),
        scratch_types=[
            pltpu.VMEM((CHUNKS, CB), jnp.int32),
            pltpu.VMEM((CHUNKS, CB), jnp.int32),
            pltpu.VMEM((CHUNKS, CB), jnp.int32),
            pltpu.VMEM((CHUNKS, CB), jnp.int32),
            pltpu.VMEM((CHUNKS, CB), jnp.float32),
            pltpu.VMEM((CHUNKS, CB), jnp.float32),
            pltpu.VMEM_SHARED((2 * GK,), jnp.float32),
        ],
    )(_sc_stage_body)


def _stage_c_body(a_ref, cb_ref, p_ref, q_ref, cnt_ref, loss_ref, w_ref, b_ref,
                  logit_ref, cg_ref, sg_ref, lo_ref):
    a_s = a_ref[0, 0] + a_ref[1, 0]
    a_n = a_ref[0, 1] + a_ref[1, 1]
    cb = cb_ref[...]

    def ndot(lhs, rhs):
        return lax.dot_general(lhs, rhs, (((1,), (0,)), ((), ())),
                               preferred_element_type=jnp.float32)

    ch, cm, cl = _split3(cb)
    sh, sm, sl = _split3(a_s)
    nh, nm, nl = _split3(a_n)
    r = (ndot(sh, cl) + ndot(sm, cm) + ndot(sl, ch)
         + ndot(sm, ch) + ndot(sh, cm) + ndot(sh, ch))
    s = (ndot(nh, cl) + ndot(nm, cm) + ndot(nl, ch)
         + ndot(nm, ch) + ndot(nh, cm) + ndot(nh, ch))
    cnt = jnp.maximum(cnt_ref[...], 1.0)
    p = p_ref[...]
    cr = p + r
    cg = cr / cnt
    sg = (q_ref[...] + s - cr) / cnt
    cg_ref[...] = cg
    sg_ref[...] = sg
    logit_ref[...] = lax.dot_general(cg, w_ref[...], (((1,), (0,)), ((), ())),
                                     preferred_element_type=jnp.float32) + b_ref[...]
    lo_ref[...] = loss_ref[...] * (CW / (N * EMB))


_stage_c = pl.pallas_call(
    _stage_c_body,
    out_shape=[
        jax.ShapeDtypeStruct((G, 128), jnp.float32),
        jax.ShapeDtypeStruct((G, EMB), jnp.float32),
        jax.ShapeDtypeStruct((G, EMB), jnp.float32),
        jax.ShapeDtypeStruct((1, 1), jnp.float32),
    ],
)


def kernel(node_feat, score, batch, codebook, W, b):
    batch = batch.astype(jnp.int32)
    score_f = score[:, 0].astype(jnp.float32)

    xp = jnp.zeros((N1, EMB), jnp.float32).at[:N].set(node_feat)
    sv = (jnp.zeros((N1, 2), jnp.float32)
          .at[:N, 0].set(score_f).at[:N, 1].set(1.0))
    bt = jnp.zeros((N1, 1), jnp.int32).at[:N, 0].set(batch)
    idx, p_sum, q_sum, cnt, loss = _stage_a(xp, sv, bt, codebook)

    bt2 = jnp.zeros((N2,), jnp.int32).at[:N].set(batch)
    ix2 = jnp.zeros((N2,), jnp.int32).at[:N1].set(idx[:, 0])
    sw2 = jnp.zeros((N2,), jnp.float32).at[:N].set(score_f)
    vw2 = jnp.zeros((N2,), jnp.float32).at[:N].set(1.0)
    zz = jnp.zeros((STRIPE,), jnp.float32)
    a_mats = _build_sc_stage()(bt2.reshape(NTILES, CHUNKS, CB),
                       ix2.reshape(NTILES, CHUNKS, CB),
                       sw2.reshape(NTILES, CHUNKS, CB),
                       vw2.reshape(NTILES, CHUNKS, CB),
                       zz)

    w_pad = jnp.zeros((EMB, 128), jnp.float32).at[:, :NC].set(W)
    b_pad = jnp.zeros((1, 128), jnp.float32).at[0, :NC].set(b)
    logit_pad, c_graph, s_graph, lo = _stage_c(
        a_mats.reshape(2, 2, G, K), codebook, p_sum, q_sum, cnt, loss,
        w_pad, b_pad)
    return (logit_pad[:, :NC], c_graph, s_graph, lo[0, 0])


# speedup vs baseline: 3.6736x; 1.4879x over previous
import functools

import jax
import jax.numpy as jnp
from jax import lax
from jax.experimental import pallas as pl
from jax.experimental.pallas import tpu as pltpu
from jax.experimental.pallas import tpu_sc as plsc

N = 10000
EMB = 256
K = 1024
G = 128
NC = 10
CW = 1.0

BN = 1000
NBLK = N // BN

NTILES = 32
CHUNKS = 3
CB = 128
PER_TILE = CHUNKS * CB
N2 = NTILES * PER_TILE
GK = G * K
STRIPE = 2 * GK // 16


def _split3(v):
    h = v.astype(jnp.bfloat16)
    r = v - h.astype(jnp.float32)
    mid = r.astype(jnp.bfloat16)
    low = (r - mid.astype(jnp.float32)).astype(jnp.bfloat16)
    return h, mid, low


def _split2(v):
    h = v.astype(jnp.bfloat16)
    mid = (v - h.astype(jnp.float32)).astype(jnp.bfloat16)
    return h, mid


def _stage_a_body(x_ref, sc_ref, bt_ref, cb_ref,
                  idx_ref, p_ref, q_ref, cnt_ref, loss_ref):
    pid = pl.program_id(0)

    @pl.when(pid == 0)
    def _init():
        p_ref[...] = jnp.zeros_like(p_ref)
        q_ref[...] = jnp.zeros_like(q_ref)
        cnt_ref[...] = jnp.zeros_like(cnt_ref)
        loss_ref[...] = jnp.zeros_like(loss_ref)

    x = x_ref[...]
    cb = cb_ref[...]
    xc = lax.dot_general(x, cb, (((1,), (1,)), ((), ())),
                         preferred_element_type=jnp.float32)
    x2 = jnp.sum(x * x, axis=1, keepdims=True)
    csh, csm, csl = _split3(cb * cb)
    ones_row = jnp.ones((1, EMB), jnp.bfloat16)

    def odot(rhs):
        return lax.dot_general(ones_row, rhs, (((1,), (1,)), ((), ())),
                               preferred_element_type=jnp.float32)

    c2 = odot(csl) + odot(csm) + odot(csh)
    d = x2 - 2.0 * xc + c2
    m = jnp.min(d, axis=1, keepdims=True)
    kio = lax.broadcasted_iota(jnp.int32, (BN, K), 1)
    a = jnp.min(jnp.where(d == m, kio, K), axis=1, keepdims=True)
    idx_ref[...] = a

    score = sc_ref[...]
    ohb = (lax.broadcasted_iota(jnp.int32, (BN, G), 1)
           == bt_ref[...]).astype(jnp.bfloat16)

    def tdot(lhs, rhs):
        return lax.dot_general(lhs, rhs, (((0,), (0,)), ((), ())),
                               preferred_element_type=jnp.float32)

    xh, xm = _split2(x)
    sh, sm = _split2(score)
    p_ref[...] += (tdot(ohb * sm, xh) + tdot(ohb * sh, xm)
                   + tdot(ohb * sh, xh))
    q_ref[...] += tdot(ohb, xm) + tdot(ohb, xh)
    cnt_ref[...] += tdot(ohb, jnp.ones((BN, 1), jnp.bfloat16))
    loss_ref[...] += jnp.sum(m).reshape(1, 1)


_stage_a = pl.pallas_call(
    _stage_a_body,
    grid=(NBLK,),
    in_specs=[
        pl.BlockSpec((BN, EMB), lambda i: (i, 0)),
        pl.BlockSpec((BN, 1), lambda i: (i, 0)),
        pl.BlockSpec((BN, 1), lambda i: (i, 0)),
        pl.BlockSpec((K, EMB), lambda i: (0, 0)),
    ],
    out_specs=[
        pl.BlockSpec((BN, 1), lambda i: (i, 0)),
        pl.BlockSpec((G, EMB), lambda i: (0, 0)),
        pl.BlockSpec((G, EMB), lambda i: (0, 0)),
        pl.BlockSpec((G, 1), lambda i: (0, 0)),
        pl.BlockSpec((1, 1), lambda i: (0, 0)),
    ],
    out_shape=[
        jax.ShapeDtypeStruct((N, 1), jnp.int32),
        jax.ShapeDtypeStruct((G, EMB), jnp.float32),
        jax.ShapeDtypeStruct((G, EMB), jnp.float32),
        jax.ShapeDtypeStruct((G, 1), jnp.float32),
        jax.ShapeDtypeStruct((1, 1), jnp.float32),
    ],
    compiler_params=pltpu.CompilerParams(dimension_semantics=("arbitrary",)),
)


def _sc_stage_body(bt_hbm, ix_hbm, sw_hbm, vw_hbm, z_hbm, out_hbm,
                   bv, kv, fi, fi2, sw, vw, acc):
    cid = lax.axis_index("c")
    sid = lax.axis_index("s")
    tid = cid * 16 + sid
    pltpu.sync_copy(z_hbm, acc.at[pl.ds(sid * STRIPE, STRIPE)])
    pltpu.sync_copy(bt_hbm.at[tid], bv)
    pltpu.sync_copy(ix_hbm.at[tid], kv)
    pltpu.sync_copy(sw_hbm.at[tid], sw)
    pltpu.sync_copy(vw_hbm.at[tid], vw)
    for j in range(CHUNKS):
        for l in range(CB // 16):
            s_ = pl.ds(l * 16, 16)
            f = bv[j, s_] * K + kv[j, s_]
            fi[j, s_] = f
            fi2[j, s_] = f + GK
    plsc.subcore_barrier()
    for j in range(CHUNKS):
        pltpu.sync_copy(sw.at[j], acc.at[fi.at[j]], add=True)
        pltpu.sync_copy(vw.at[j], acc.at[fi2.at[j]], add=True)
    plsc.subcore_barrier()
    pltpu.sync_copy(acc.at[pl.ds(sid * STRIPE, STRIPE)],
                    out_hbm.at[cid, pl.ds(sid * STRIPE, STRIPE)])


@functools.cache
def _build_sc_stage():
    return functools.partial(
        pl.kernel,
        mesh=plsc.VectorSubcoreMesh(core_axis_name="c", subcore_axis_name="s"),
        out_type=jax.ShapeDtypeStruct((2, 2 * GK), jnp.float32),
        scratch_types=[
            pltpu.VMEM((CHUNKS, CB), jnp.int32),
            pltpu.VMEM((CHUNKS, CB), jnp.int32),
            pltpu.VMEM((CHUNKS, CB), jnp.int32),
            pltpu.VMEM((CHUNKS, CB), jnp.int32),
            pltpu.VMEM((CHUNKS, CB), jnp.float32),
            pltpu.VMEM((CHUNKS, CB), jnp.float32),
            pltpu.VMEM_SHARED((2 * GK,), jnp.float32),
        ],
    )(_sc_stage_body)


def _stage_c_body(a_ref, cb_ref, p_ref, q_ref, cnt_ref, loss_ref, w_ref, b_ref,
                  logit_ref, cg_ref, sg_ref, lo_ref):
    a_s = a_ref[0, 0] + a_ref[1, 0]
    a_n = a_ref[0, 1] + a_ref[1, 1]
    cb = cb_ref[...]

    def ndot(lhs, rhs):
        return lax.dot_general(lhs, rhs, (((1,), (0,)), ((), ())),
                               preferred_element_type=jnp.float32)

    ch, cm = _split2(cb)
    sh, sm = _split2(a_s)
    nh, nm = _split2(a_n)
    r = ndot(sm, ch) + ndot(sh, cm) + ndot(sh, ch)
    s = ndot(nm, ch) + ndot(nh, cm) + ndot(nh, ch)
    cnt = jnp.maximum(cnt_ref[...], 1.0)
    p = p_ref[...]
    cr = p + r
    cg = cr / cnt
    sg = (q_ref[...] + s - cr) / cnt
    cg_ref[...] = cg
    sg_ref[...] = sg
    logit_ref[...] = lax.dot_general(cg, w_ref[...], (((1,), (0,)), ((), ())),
                                     preferred_element_type=jnp.float32) + b_ref[...]
    lo_ref[...] = loss_ref[...] * (CW / (N * EMB))


_stage_c = pl.pallas_call(
    _stage_c_body,
    out_shape=[
        jax.ShapeDtypeStruct((G, 128), jnp.float32),
        jax.ShapeDtypeStruct((G, EMB), jnp.float32),
        jax.ShapeDtypeStruct((G, EMB), jnp.float32),
        jax.ShapeDtypeStruct((1, 1), jnp.float32),
    ],
)


def kernel(node_feat, score, batch, codebook, W, b):
    batch = batch.astype(jnp.int32)

    idx, p_sum, q_sum, cnt, loss = _stage_a(
        node_feat, score, batch[:, None], codebook)

    bt2 = jnp.zeros((N2,), jnp.int32).at[:N].set(batch)
    ix2 = jnp.zeros((N2,), jnp.int32).at[:N].set(idx[:, 0])
    sw2 = jnp.zeros((N2,), jnp.float32).at[:N].set(score[:, 0])
    vw2 = jnp.zeros((N2,), jnp.float32).at[:N].set(1.0)
    zz = jnp.zeros((STRIPE,), jnp.float32)
    a_mats = _build_sc_stage()(bt2.reshape(NTILES, CHUNKS, CB),
                               ix2.reshape(NTILES, CHUNKS, CB),
                               sw2.reshape(NTILES, CHUNKS, CB),
                               vw2.reshape(NTILES, CHUNKS, CB),
                               zz)

    w_pad = jnp.zeros((EMB, 128), jnp.float32).at[:, :NC].set(W)
    b_pad = jnp.zeros((1, 128), jnp.float32).at[0, :NC].set(b)
    logit_pad, c_graph, s_graph, lo = _stage_c(
        a_mats.reshape(2, 2, G, K), codebook, p_sum, q_sum, cnt, loss,
        w_pad, b_pad)
    return (logit_pad[:, :NC], c_graph, s_graph, lo[0, 0])
